# baseline (speedup 1.0000x reference)
import functools

import jax
import jax.numpy as jnp
from jax import lax
from jax.experimental import pallas as pl
from jax.experimental.pallas import tpu as pltpu
from jax.experimental.pallas import tpu_sc as plsc


_TILE = 512


def _add_body(x_ref, pos_ref, out_ref):
    out_ref[...] = x_ref[...] + pos_ref[...][None, :, :]


def _kernel_tc(x, pos_table):
    batch, seq_len, embed_dim = x.shape
    grid = (seq_len // _TILE,)
    return pl.pallas_call(
        _add_body,
        grid=grid,
        in_specs=[
            pl.BlockSpec((batch, _TILE, embed_dim), lambda i: (0, i, 0)),
            pl.BlockSpec((_TILE, embed_dim), lambda i: (i, 0)),
        ],
        out_specs=pl.BlockSpec((batch, _TILE, embed_dim), lambda i: (0, i, 0)),
        out_shape=jax.ShapeDtypeStruct(x.shape, x.dtype),
    )(x, pos_table[:seq_len])


_CHW = 16384
_LANES = 16


def _kernel_sc(x, pos_table):
    batch, seq_len, embed_dim = x.shape
    info = plsc.get_sparse_core_info()
    nc, ns = info.num_cores, info.num_subcores
    nw = nc * ns
    words = seq_len * embed_dim
    wpw = words // nw
    nch = wpw // _CHW
    mesh = plsc.VectorSubcoreMesh(core_axis_name="c", subcore_axis_name="s")

    @functools.partial(
        pl.kernel,
        mesh=mesh,
        out_type=jax.ShapeDtypeStruct((batch, words), x.dtype),
        scratch_types=[
            pltpu.VMEM((_CHW,), jnp.float32),
            pltpu.VMEM((_CHW,), jnp.float32),
        ],
    )
    def sc_add(x_hbm, pos_hbm, out_hbm, pos_v, x_v):
        wid = lax.axis_index("s") * nc + lax.axis_index("c")
        base = wid * wpw

        def chunk_body(c, carry):
            off = base + c * _CHW
            pltpu.sync_copy(pos_hbm.at[pl.ds(off, _CHW)], pos_v)
            for b in range(batch):
                pltpu.sync_copy(x_hbm.at[b, pl.ds(off, _CHW)], x_v)

                def vec_body(j, carry2):
                    sl = pl.ds(j * _LANES, _LANES)
                    x_v[sl] = x_v[sl] + pos_v[sl]
                    return carry2

                lax.fori_loop(0, _CHW // _LANES, vec_body, 0, unroll=8)
                pltpu.sync_copy(x_v, out_hbm.at[b, pl.ds(off, _CHW)])
            return carry

        lax.fori_loop(0, nch, chunk_body, 0)

    out2 = sc_add(x.reshape(batch, words), pos_table[:seq_len].reshape(words))
    return out2.reshape(batch, seq_len, embed_dim)


def kernel(x, pos_table):
    return _kernel_sc(x, pos_table)


# speedup vs baseline: 1.4318x; 1.4318x over previous
import functools

import jax
import jax.numpy as jnp
from jax import lax
from jax.experimental import pallas as pl
from jax.experimental.pallas import tpu as pltpu
from jax.experimental.pallas import tpu_sc as plsc


_TILE = 512


def _add_body(x_ref, pos_ref, out_ref):
    out_ref[...] = x_ref[...] + pos_ref[...][None, :, :]


def _kernel_tc(x, pos_table):
    batch, seq_len, embed_dim = x.shape
    grid = (seq_len // _TILE,)
    return pl.pallas_call(
        _add_body,
        grid=grid,
        in_specs=[
            pl.BlockSpec((batch, _TILE, embed_dim), lambda i: (0, i, 0)),
            pl.BlockSpec((_TILE, embed_dim), lambda i: (i, 0)),
        ],
        out_specs=pl.BlockSpec((batch, _TILE, embed_dim), lambda i: (0, i, 0)),
        out_shape=jax.ShapeDtypeStruct(x.shape, x.dtype),
    )(x, pos_table[:seq_len])


_CHW = 16384
_LANES = 16


_CHROWS = 16


def _kernel_sc(x, pos_table):
    batch, seq_len, embed_dim = x.shape
    info = plsc.get_sparse_core_info()
    nc, ns = info.num_cores, info.num_subcores
    nw = nc * ns
    rpw = seq_len // nw
    nch = rpw // _CHROWS
    cols = embed_dim // _LANES
    mesh = plsc.VectorSubcoreMesh(core_axis_name="c", subcore_axis_name="s")

    @functools.partial(
        pl.kernel,
        mesh=mesh,
        out_type=jax.ShapeDtypeStruct(x.shape, x.dtype),
        scratch_types=[
            pltpu.VMEM((_CHROWS, embed_dim), jnp.float32),
            pltpu.VMEM((_CHROWS, embed_dim), jnp.float32),
        ],
    )
    def sc_add(x_hbm, pos_hbm, out_hbm, pos_v, x_v):
        wid = lax.axis_index("s") * nc + lax.axis_index("c")
        base = wid * rpw

        def chunk_body(c, carry):
            off = base + c * _CHROWS
            pltpu.sync_copy(pos_hbm.at[pl.ds(off, _CHROWS)], pos_v)
            for b in range(batch):
                pltpu.sync_copy(x_hbm.at[b, pl.ds(off, _CHROWS)], x_v)

                def row_body(i, carry2):
                    def vec_body(j, carry3):
                        sl = pl.ds(j * _LANES, _LANES)
                        x_v[i, sl] = x_v[i, sl] + pos_v[i, sl]
                        return carry3

                    return lax.fori_loop(0, cols, vec_body, carry2, unroll=8)

                lax.fori_loop(0, _CHROWS, row_body, 0)
                pltpu.sync_copy(x_v, out_hbm.at[b, pl.ds(off, _CHROWS)])
            return carry

        lax.fori_loop(0, nch, chunk_body, 0)

    return sc_add(x, pos_table[:seq_len])


def kernel(x, pos_table):
    return _kernel_sc(x, pos_table)


# speedup vs baseline: 5.4058x; 3.7755x over previous
import functools

import jax
import jax.numpy as jnp
from jax import lax
from jax.experimental import pallas as pl
from jax.experimental.pallas import tpu as pltpu
from jax.experimental.pallas import tpu_sc as plsc


_TILE = 512


def _add_body(x_ref, pos_ref, out_ref):
    out_ref[...] = x_ref[...] + pos_ref[...][None, :, :]


def _kernel_tc(x, pos_table):
    batch, seq_len, embed_dim = x.shape
    grid = (seq_len // _TILE,)
    return pl.pallas_call(
        _add_body,
        grid=grid,
        in_specs=[
            pl.BlockSpec((batch, _TILE, embed_dim), lambda i: (0, i, 0)),
            pl.BlockSpec((_TILE, embed_dim), lambda i: (i, 0)),
        ],
        out_specs=pl.BlockSpec((batch, _TILE, embed_dim), lambda i: (0, i, 0)),
        out_shape=jax.ShapeDtypeStruct(x.shape, x.dtype),
    )(x, pos_table[:seq_len])


_CHW = 16384
_LANES = 16


_CHROWS = 8


def _kernel_sc(x, pos_table):
    batch, seq_len, embed_dim = x.shape
    info = plsc.get_sparse_core_info()
    nc, ns = info.num_cores, info.num_subcores
    nw = nc * ns
    rpw = seq_len // nw
    nch = rpw // _CHROWS
    cols = embed_dim // _LANES
    mesh = plsc.VectorSubcoreMesh(core_axis_name="c", subcore_axis_name="s")

    n_sems = 2 * batch + 2 * batch + 2

    @functools.partial(
        pl.kernel,
        mesh=mesh,
        out_type=jax.ShapeDtypeStruct(x.shape, x.dtype),
        scratch_types=[
            pltpu.VMEM((2, batch, _CHROWS, embed_dim), jnp.float32),
            pltpu.VMEM((2, _CHROWS, embed_dim), jnp.float32),
        ]
        + [pltpu.SemaphoreType.DMA] * n_sems,
    )
    def sc_add(x_hbm, pos_hbm, out_hbm, x_s, pos_s, *sems):
        in_sems = [list(sems[0:batch]), list(sems[batch : 2 * batch])]
        out_sems = [
            list(sems[2 * batch : 3 * batch]),
            list(sems[3 * batch : 4 * batch]),
        ]
        pos_sems = list(sems[4 * batch : 4 * batch + 2])
        wid = lax.axis_index("s") * nc + lax.axis_index("c")
        base = wid * rpw

        def wait_x(slot_ref, sem):
            pltpu.make_async_copy(x_hbm.at[0, pl.ds(0, _CHROWS)], slot_ref, sem).wait()

        def wait_pos(slot_ref, sem):
            pltpu.make_async_copy(pos_hbm.at[pl.ds(0, _CHROWS)], slot_ref, sem).wait()

        pltpu.async_copy(pos_hbm.at[pl.ds(base, _CHROWS)], pos_s.at[0], pos_sems[0])
        for b in range(batch):
            pltpu.async_copy(
                x_hbm.at[b, pl.ds(base, _CHROWS)], x_s.at[0, b], in_sems[0][b]
            )

        def pair_body(p, carry):
            for cp in (0, 1):
                c = 2 * p + cp
                off = base + c * _CHROWS
                @pl.when(c + 1 < nch)
                def _():
                    pltpu.async_copy(
                        pos_hbm.at[pl.ds(off + _CHROWS, _CHROWS)],
                        pos_s.at[cp ^ 1],
                        pos_sems[cp ^ 1],
                    )

                wait_pos(pos_s.at[cp], pos_sems[cp])
                for b in range(batch):
                    wait_x(x_s.at[cp, b], in_sems[cp][b])

                    def row_body(i, carry2, b=b, cp=cp):
                        def vec_body(j, carry3, i=i):
                            sl = pl.ds(j * _LANES, _LANES)
                            plsc.addupdate(x_s.at[cp, b, i, sl], pos_s[cp, i, sl])
                            return carry3

                        return lax.fori_loop(0, cols, vec_body, carry2, unroll=8)

                    lax.fori_loop(0, _CHROWS, row_body, 0)
                    pltpu.async_copy(
                        x_s.at[cp, b], out_hbm.at[b, pl.ds(off, _CHROWS)], out_sems[cp][b]
                    )

                    @pl.when(c + 1 < nch)
                    def _(b=b, cp=cp, off=off, c=c):
                        @pl.when(c > 0)
                        def _():
                            wait_x(x_s.at[cp ^ 1, b], out_sems[cp ^ 1][b])

                        pltpu.async_copy(
                            x_hbm.at[b, pl.ds(off + _CHROWS, _CHROWS)],
                            x_s.at[cp ^ 1, b],
                            in_sems[cp ^ 1][b],
                        )

            return carry

        lax.fori_loop(0, nch // 2, pair_body, 0)

        last_cp = (nch - 1) & 1
        for b in range(batch):
            wait_x(x_s.at[last_cp, b], out_sems[last_cp][b])

    return sc_add(x, pos_table[:seq_len])


def kernel(x, pos_table):
    return _kernel_sc(x, pos_table)


# speedup vs baseline: 5.6453x; 1.0443x over previous
import functools

import jax
import jax.numpy as jnp
from jax import lax
from jax.experimental import pallas as pl
from jax.experimental.pallas import tpu as pltpu
from jax.experimental.pallas import tpu_sc as plsc


_TILE = 512


def _add_body(x_ref, pos_ref, out_ref):
    out_ref[...] = x_ref[...] + pos_ref[...][None, :, :]


def _kernel_tc(x, pos_table):
    batch, seq_len, embed_dim = x.shape
    grid = (seq_len // _TILE,)
    return pl.pallas_call(
        _add_body,
        grid=grid,
        in_specs=[
            pl.BlockSpec((batch, _TILE, embed_dim), lambda i: (0, i, 0)),
            pl.BlockSpec((_TILE, embed_dim), lambda i: (i, 0)),
        ],
        out_specs=pl.BlockSpec((batch, _TILE, embed_dim), lambda i: (0, i, 0)),
        out_shape=jax.ShapeDtypeStruct(x.shape, x.dtype),
    )(x, pos_table[:seq_len])


_CHW = 16384
_LANES = 16


_CHROWS = 8


def _kernel_sc(x, pos_table):
    batch, seq_len, embed_dim = x.shape
    info = plsc.get_sparse_core_info()
    nc, ns = info.num_cores, info.num_subcores
    nw = nc * ns
    rpw = seq_len // nw
    nch = rpw // _CHROWS
    cols = embed_dim // _LANES
    mesh = plsc.VectorSubcoreMesh(core_axis_name="c", subcore_axis_name="s")

    n_sems = 2 * batch + 2 * batch + 2

    @functools.partial(
        pl.kernel,
        mesh=mesh,
        out_type=jax.ShapeDtypeStruct(x.shape, x.dtype),
        scratch_types=[
            pltpu.VMEM((2, batch, _CHROWS, embed_dim), jnp.float32),
            pltpu.VMEM((2, _CHROWS, embed_dim), jnp.float32),
        ]
        + [pltpu.SemaphoreType.DMA] * n_sems,
    )
    def sc_add(x_hbm, pos_hbm, out_hbm, x_s, pos_s, *sems):
        in_sems = [list(sems[0:batch]), list(sems[batch : 2 * batch])]
        out_sems = [
            list(sems[2 * batch : 3 * batch]),
            list(sems[3 * batch : 4 * batch]),
        ]
        pos_sems = list(sems[4 * batch : 4 * batch + 2])
        wid = lax.axis_index("s") * nc + lax.axis_index("c")
        base = wid * rpw

        def wait_x(slot_ref, sem):
            pltpu.make_async_copy(x_hbm.at[0, pl.ds(0, _CHROWS)], slot_ref, sem).wait()

        def wait_pos(slot_ref, sem):
            pltpu.make_async_copy(pos_hbm.at[pl.ds(0, _CHROWS)], slot_ref, sem).wait()

        pltpu.async_copy(pos_hbm.at[pl.ds(base, _CHROWS)], pos_s.at[0], pos_sems[0])
        for b in range(batch):
            pltpu.async_copy(
                x_hbm.at[b, pl.ds(base, _CHROWS)], x_s.at[0, b], in_sems[0][b]
            )

        def pair_body(p, carry):
            for cp in (0, 1):
                c = 2 * p + cp
                off = base + c * _CHROWS
                @pl.when(c + 1 < nch)
                def _():
                    pltpu.async_copy(
                        pos_hbm.at[pl.ds(off + _CHROWS, _CHROWS)],
                        pos_s.at[cp ^ 1],
                        pos_sems[cp ^ 1],
                    )

                wait_pos(pos_s.at[cp], pos_sems[cp])
                for b in range(batch):
                    wait_x(x_s.at[cp, b], in_sems[cp][b])

                    col_shift = cols.bit_length() - 1

                    @plsc.parallel_loop(0, _CHROWS * cols, unroll=8)
                    def _(g, b=b, cp=cp):
                        i = g >> col_shift
                        sl = pl.ds((g & (cols - 1)) * _LANES, _LANES)
                        plsc.addupdate(x_s.at[cp, b, i, sl], pos_s[cp, i, sl])
                    pltpu.async_copy(
                        x_s.at[cp, b], out_hbm.at[b, pl.ds(off, _CHROWS)], out_sems[cp][b]
                    )

                    @pl.when(c + 1 < nch)
                    def _(b=b, cp=cp, off=off, c=c):
                        @pl.when(c > 0)
                        def _():
                            wait_x(x_s.at[cp ^ 1, b], out_sems[cp ^ 1][b])

                        pltpu.async_copy(
                            x_hbm.at[b, pl.ds(off + _CHROWS, _CHROWS)],
                            x_s.at[cp ^ 1, b],
                            in_sems[cp ^ 1][b],
                        )

            return carry

        lax.fori_loop(0, nch // 2, pair_body, 0)

        last_cp = (nch - 1) & 1
        for b in range(batch):
            wait_x(x_s.at[last_cp, b], out_sems[last_cp][b])

    return sc_add(x, pos_table[:seq_len])


def kernel(x, pos_table):
    return _kernel_sc(x, pos_table)
